# baseline (speedup 1.0000x reference)
import functools

import jax
import jax.numpy as jnp
from jax import lax
from jax.experimental import pallas as pl
from jax.experimental.pallas import tpu as pltpu
from jax.experimental.pallas import tpu_sc as plsc

_ROWS = 16


def _fold_table(wte, W, b):
    def body(wte_ref, w_ref, b_ref, out_ref):
        out_ref[...] = lax.dot_general(
            wte_ref[...], w_ref[...],
            (((1,), (1,)), ((), ())),
            preferred_element_type=jnp.float32) + b_ref[...]

    return pl.pallas_call(
        body,
        out_shape=jax.ShapeDtypeStruct(wte.shape, jnp.float32),
    )(wte, W, b.reshape(1, -1))


def _sc_gather(idx, table):
    bsz, seq = idx.shape
    d = table.shape[1]
    info = plsc.get_sparse_core_info()
    nw = info.num_cores * info.num_subcores
    rpw = bsz // nw
    nchunks = rpw // _ROWS

    mesh = plsc.VectorSubcoreMesh(core_axis_name="c", subcore_axis_name="s")

    @functools.partial(
        pl.kernel,
        out_type=jax.ShapeDtypeStruct((bsz, seq, d), jnp.float32),
        mesh=mesh,
        scratch_types=[
            pltpu.VMEM((2, _ROWS, seq), jnp.int32),
            pltpu.VMEM((2, _ROWS, seq, d), jnp.float32),
            pltpu.SemaphoreType.DMA,
            pltpu.SemaphoreType.DMA,
            pltpu.SemaphoreType.DMA,
        ],
        compiler_params=pltpu.CompilerParams(use_tc_tiling_on_sc=False),
    )
    def k(idx_hbm, table_hbm, out_hbm, idx_v, rows_v, sem_i0, sem_i1, sem_g):
        wid = lax.axis_index("s") * info.num_cores + lax.axis_index("c")
        row0 = wid * rpw
        sem_i = (sem_i0, sem_i1)

        def idx_copy(c, buf):
            return pltpu.make_async_copy(
                idx_hbm.at[pl.ds(row0 + c * _ROWS, _ROWS)],
                idx_v.at[buf], sem_i[buf])

        idx_copy(0, 0).start()

        def pair(cc, carry):
            for par in (0, 1):
                c = cc * 2 + par

                @pl.when(c + 1 < nchunks)
                def _():
                    idx_copy(c + 1, 1 - par).start()

                idx_copy(c, par).wait()
                copies = [
                    pltpu.async_copy(table_hbm.at[idx_v.at[par, j]],
                                     rows_v.at[par, j], sem_g)
                    for j in range(_ROWS)
                ]
                for cp in copies:
                    cp.wait()
                b0 = row0 + c * _ROWS
                pltpu.sync_copy(rows_v.at[par], out_hbm.at[pl.ds(b0, _ROWS)])
            return carry

        lax.fori_loop(0, nchunks // 2, pair, 0)

    return k(idx, table)


def kernel(x, wte, W, b):
    table2 = _fold_table(wte, W, b)
    return _sc_gather(x.astype(jnp.int32), table2)


# speedup vs baseline: 1.1353x; 1.1353x over previous
import functools

import jax
import jax.numpy as jnp
from jax import lax
from jax.experimental import pallas as pl
from jax.experimental.pallas import tpu as pltpu
from jax.experimental.pallas import tpu_sc as plsc

_ROWS = 16


def _fold_table(wte, W, b):
    def body(wte_ref, w_ref, b_ref, out_ref):
        out_ref[...] = lax.dot_general(
            wte_ref[...], w_ref[...],
            (((1,), (1,)), ((), ())),
            preferred_element_type=jnp.float32) + b_ref[...]

    return pl.pallas_call(
        body,
        out_shape=jax.ShapeDtypeStruct(wte.shape, jnp.float32),
    )(wte, W, b.reshape(1, -1))


def _sc_gather(idx, table):
    bsz, seq = idx.shape
    d = table.shape[1]
    info = plsc.get_sparse_core_info()
    nw = info.num_cores * info.num_subcores
    rpw = bsz // nw
    nchunks = rpw // _ROWS

    mesh = plsc.VectorSubcoreMesh(core_axis_name="c", subcore_axis_name="s")

    @functools.partial(
        pl.kernel,
        out_type=jax.ShapeDtypeStruct((bsz, seq, d), jnp.float32),
        mesh=mesh,
        scratch_types=[
            pltpu.VMEM((2, _ROWS, seq), jnp.int32),
            pltpu.VMEM((2, _ROWS, seq, d), jnp.float32),
            pltpu.VMEM_SHARED(table.shape, jnp.float32),
            pltpu.SemaphoreType.DMA,
            pltpu.SemaphoreType.DMA,
            pltpu.SemaphoreType.DMA,
        ],
        compiler_params=pltpu.CompilerParams(use_tc_tiling_on_sc=False),
    )
    def k(idx_hbm, table_hbm, out_hbm, idx_v, rows_v, table_v,
          sem_i0, sem_i1, sem_g):
        wid = lax.axis_index("s") * info.num_cores + lax.axis_index("c")
        row0 = wid * rpw
        sem_i = (sem_i0, sem_i1)
        pltpu.sync_copy(table_hbm, table_v)

        def idx_copy(c, buf):
            return pltpu.make_async_copy(
                idx_hbm.at[pl.ds(row0 + c * _ROWS, _ROWS)],
                idx_v.at[buf], sem_i[buf])

        idx_copy(0, 0).start()

        def pair(cc, carry):
            for par in (0, 1):
                c = cc * 2 + par

                @pl.when(c + 1 < nchunks)
                def _():
                    idx_copy(c + 1, 1 - par).start()

                idx_copy(c, par).wait()
                copies = [
                    pltpu.async_copy(table_v.at[idx_v.at[par, j]],
                                     rows_v.at[par, j], sem_g)
                    for j in range(_ROWS)
                ]
                for cp in copies:
                    cp.wait()
                b0 = row0 + c * _ROWS
                pltpu.sync_copy(rows_v.at[par], out_hbm.at[pl.ds(b0, _ROWS)])
            return carry

        lax.fori_loop(0, nchunks // 2, pair, 0)

    return k(idx, table)


def kernel(x, wte, W, b):
    table2 = _fold_table(wte, W, b)
    return _sc_gather(x.astype(jnp.int32), table2)


# speedup vs baseline: 1.1609x; 1.0225x over previous
import functools

import jax
import jax.numpy as jnp
from jax import lax
from jax.experimental import pallas as pl
from jax.experimental.pallas import tpu as pltpu
from jax.experimental.pallas import tpu_sc as plsc

_ROWS = 16


def _fold_table(wte, W, b):
    def body(wte_ref, w_ref, b_ref, out_ref):
        out_ref[...] = lax.dot_general(
            wte_ref[...], w_ref[...],
            (((1,), (1,)), ((), ())),
            preferred_element_type=jnp.float32) + b_ref[...]

    return pl.pallas_call(
        body,
        out_shape=jax.ShapeDtypeStruct(wte.shape, jnp.float32),
    )(wte, W, b.reshape(1, -1))


def _sc_gather(idx, table):
    bsz, seq = idx.shape
    d = table.shape[1]
    info = plsc.get_sparse_core_info()
    nw = info.num_cores * info.num_subcores
    rpw = bsz // nw
    nchunks = rpw // _ROWS

    mesh = plsc.VectorSubcoreMesh(core_axis_name="c", subcore_axis_name="s")

    @functools.partial(
        pl.kernel,
        out_type=jax.ShapeDtypeStruct((bsz, seq, d), jnp.float32),
        mesh=mesh,
        scratch_types=[
            pltpu.VMEM((2, _ROWS, seq), jnp.int32),
            pltpu.VMEM((2, _ROWS, seq, d), jnp.float32),
            pltpu.VMEM_SHARED(table.shape, jnp.float32),
            pltpu.SemaphoreType.DMA,
            pltpu.SemaphoreType.DMA,
            pltpu.SemaphoreType.DMA,
            pltpu.SemaphoreType.DMA,
            pltpu.SemaphoreType.DMA,
            pltpu.SemaphoreType.DMA,
        ],
        compiler_params=pltpu.CompilerParams(use_tc_tiling_on_sc=False),
    )
    def k(idx_hbm, table_hbm, out_hbm, idx_v, rows_v, table_v,
          sem_i0, sem_i1, sem_g0, sem_g1, sem_o0, sem_o1):
        wid = lax.axis_index("s") * info.num_cores + lax.axis_index("c")
        row0 = wid * rpw
        sem_i = (sem_i0, sem_i1)
        sem_g = (sem_g0, sem_g1)
        sem_o = (sem_o0, sem_o1)
        pltpu.sync_copy(table_hbm, table_v)

        def idx_copy(c, buf):
            return pltpu.make_async_copy(
                idx_hbm.at[pl.ds(row0 + c * _ROWS, _ROWS)],
                idx_v.at[buf], sem_i[buf])

        def gathers(buf):
            return [
                pltpu.make_async_copy(table_v.at[idx_v.at[buf, j]],
                                      rows_v.at[buf, j], sem_g[buf])
                for j in range(_ROWS)
            ]

        def out_copy(c, buf):
            return pltpu.make_async_copy(
                rows_v.at[buf], out_hbm.at[pl.ds(row0 + c * _ROWS, _ROWS)],
                sem_o[buf])

        idx_copy(0, 0).start()

        def pair(cc, carry):
            for par in (0, 1):
                c = cc * 2 + par

                @pl.when(c >= 2)
                def _():
                    out_copy(c - 2, par).wait()

                idx_copy(c, par).wait()
                for cp in gathers(par):
                    cp.start()

                @pl.when(c >= 1)
                def _():
                    for cp in gathers(1 - par):
                        cp.wait()
                    out_copy(c - 1, 1 - par).start()

                @pl.when(c + 1 < nchunks)
                def _():
                    idx_copy(c + 1, 1 - par).start()
            return carry

        lax.fori_loop(0, nchunks // 2, pair, 0)

        last = nchunks - 1
        for cp in gathers(last % 2):
            cp.wait()
        out_copy(last - 1, (last - 1) % 2).wait()
        out_copy(last, last % 2).start()
        out_copy(last, last % 2).wait()

    return k(idx, table)


def kernel(x, wte, W, b):
    table2 = _fold_table(wte, W, b)
    return _sc_gather(x.astype(jnp.int32), table2)
